# baseline (speedup 1.0000x reference)
import functools

import numpy as np
import jax
import jax.numpy as jnp
from jax import lax
from jax.experimental import pallas as pl
from jax.experimental.pallas import tpu as pltpu
from jax.experimental.pallas import tpu_sc as plsc

H = W = 512
B = 64
NPIX = H * W
MAXR = 256
NBINS = 512
NCOLH = 256
NPIXH = H * NCOLH


def _bitrev(n_bits):
    n = 1 << n_bits
    p = np.zeros(n, np.int64)
    for i in range(n):
        b = 0
        for k in range(n_bits):
            b |= ((i >> k) & 1) << (n_bits - 1 - k)
        p[i] = b
    return p


_PERM_NP = _bitrev(8)


def _dft_mats():
    j = np.arange(H, dtype=np.int64)
    jk = np.outer(j, j) % H
    ang = (2.0 * np.pi / H) * jk.astype(np.float64)
    fr = (np.cos(ang) / H).astype(np.float32)
    fi = (-np.sin(ang) / H).astype(np.float32)
    frh = np.ascontiguousarray(fr[:, :NCOLH][:, _PERM_NP])
    fih = np.ascontiguousarray(fi[:, :NCOLH][:, _PERM_NP])
    a = np.arange(256, dtype=np.int64)
    as2 = np.outer(a, a) % 256
    ang2 = (2.0 * np.pi / 256.0) * as2.astype(np.float64)
    f2r = (np.cos(ang2) / H).astype(np.float32)
    f2i = (-np.sin(ang2) / H).astype(np.float32)
    u = np.arange(256, dtype=np.float64)[:, None]
    wr = np.broadcast_to(np.cos(2.0 * np.pi * u / 512.0), (256, 256))
    wi = np.broadcast_to(-np.sin(2.0 * np.pi * u / 512.0), (256, 256))
    return (frh, fih, f2r, f2i,
            np.ascontiguousarray(wr).astype(np.float32),
            np.ascontiguousarray(wi).astype(np.float32))


_FRH_NP, _FIH_NP, _F2R_NP, _F2I_NP, _TWR_NP, _TWI_NP = _dft_mats()

_WCOL_NP = np.full((1, NCOLH), 2.0, np.float32)
_WCOL_NP[0, _PERM_NP == 0] = 1.0


def _radius_map():
    u = np.arange(H)
    d = ((u + H // 2) % H) - H // 2
    dy = d[:, None]
    dx = d[None, :]
    r = np.sqrt(dy * dy + dx * dx).astype(np.int64)
    return r.astype(np.int32)


_RMAP2D_NP = _radius_map()
_COUNTS_NP = np.bincount(
    _RMAP2D_NP.reshape(-1), minlength=NBINS).astype(np.float32)
_RMAPH_NP = np.ascontiguousarray(
    _RMAP2D_NP[:, :NCOLH][:, _PERM_NP]).reshape(-1)
_INVC_NP = np.zeros((1, MAXR), np.float32)
_INVC_NP[0, :] = 1.0 / _COUNTS_NP[:MAXR]

_M1_ROWS, _M1_COLS, _M1_WIDX = [], [], []
for _o in range(16):
    for _t in range(MAXR):
        for _k in range(3):
            _s = _t + _k - 1
            if 0 <= _s < MAXR:
                _M1_ROWS.append(_s)
                _M1_COLS.append(_o * MAXR + _t)
                _M1_WIDX.append(_o * 3 + _k)
_M1_ROWS = np.asarray(_M1_ROWS, np.int32)
_M1_COLS = np.asarray(_M1_COLS, np.int32)
_M1_WIDX = np.asarray(_M1_WIDX, np.int32)


def _edge_masks(rows, period):
    t = np.arange(rows) % period
    mp = (t != 0).astype(np.float32).reshape(rows, 1)
    ml = (t != period - 1).astype(np.float32).reshape(rows, 1)
    return mp, ml


_MP2_NP, _ML2_NP = _edge_masks(B * 128, 128)
_MP3_NP, _ML3_NP = _edge_masks(B * 64, 64)

_S_NP = np.zeros((B, B * 64), np.float32)
for _b in range(B):
    _S_NP[_b, _b * 64:(_b + 1) * 64] = 1.0 / 64.0


def _fft_mag_body(x_ref, frh_ref, fih_ref, f2r_ref, f2i_ref, twr_ref,
                  twi_ref, w_ref, out_ref):
    dot = functools.partial(jnp.dot, preferred_element_type=jnp.float32)
    ge = (0.2989 * x_ref[0, 0, :, 0] + 0.587 * x_ref[0, 1, :, 0]
          + 0.114 * x_ref[0, 2, :, 0])
    go = (0.2989 * x_ref[0, 0, :, 1] + 0.587 * x_ref[0, 1, :, 1]
          + 0.114 * x_ref[0, 2, :, 1])
    frh = frh_ref[...]
    fih = fih_ref[...]
    zer = dot(ge, frh)
    zei = dot(ge, fih)
    zor = dot(go, frh)
    zoi = dot(go, fih)
    f2r = f2r_ref[...]
    f2i = f2i_ref[...]
    er = dot(f2r, zer) - dot(f2i, zei)
    ei = dot(f2r, zei) + dot(f2i, zer)
    our = dot(f2r, zor) - dot(f2i, zoi)
    oui = dot(f2r, zoi) + dot(f2i, zor)
    tr = twr_ref[...] * our - twi_ref[...] * oui
    ti = twr_ref[...] * oui + twi_ref[...] * our
    w = w_ref[...]
    yrt = er + tr
    yit = ei + ti
    yrb = er - tr
    yib = ei - ti
    out_ref[0, 0:256] = jnp.sqrt(yrt * yrt + yit * yit) * w
    out_ref[0, 256:512] = jnp.sqrt(yrb * yrb + yib * yib) * w


def _fft_mag(x, frh, fih, f2r, f2i, twr, twi, wcol):
    nb = x.shape[0]
    return pl.pallas_call(
        _fft_mag_body,
        grid=(nb,),
        in_specs=[
            pl.BlockSpec((1, 3, H // 2, 2, W), lambda i: (i, 0, 0, 0, 0)),
            pl.BlockSpec((H, NCOLH), lambda i: (0, 0)),
            pl.BlockSpec((H, NCOLH), lambda i: (0, 0)),
            pl.BlockSpec((256, 256), lambda i: (0, 0)),
            pl.BlockSpec((256, 256), lambda i: (0, 0)),
            pl.BlockSpec((256, 256), lambda i: (0, 0)),
            pl.BlockSpec((256, 256), lambda i: (0, 0)),
            pl.BlockSpec((1, NCOLH), lambda i: (0, 0)),
        ],
        out_specs=pl.BlockSpec((1, H, NCOLH), lambda i: (i, 0, 0)),
        out_shape=jax.ShapeDtypeStruct((nb, H, NCOLH), jnp.float32),
        compiler_params=pltpu.CompilerParams(
            dimension_semantics=("arbitrary",)),
    )(x.reshape(x.shape[0], 3, H // 2, 2, W), frh, fih, f2r, f2i, twr, twi,
      wcol)


_NC, _NS = 2, 16
_NW = _NC * _NS
_CH = 16384
_NCHUNK = NPIXH // _CH
_IMGS_PER_W = B // _NW


def _sc_hist_body(nimg, mag_hbm, rmap_hbm, out_hbm, idx_v, *vbufs):
    wid = lax.axis_index("s") * _NC + lax.axis_index("c")
    i0 = wid * nimg
    m_v = vbufs[:nimg]
    h_v = vbufs[nimg:]

    zero = jnp.zeros((16,), jnp.float32)

    def zbody(j, carry):
        for k in range(nimg):
            h_v[k][pl.ds(j * 16, 16)] = zero
        return carry

    lax.fori_loop(0, NBINS // 16, zbody, 0)

    def cbody(c, carry):
        base = c * _CH
        pltpu.sync_copy(rmap_hbm.at[pl.ds(base, _CH)], idx_v)
        for k in range(nimg):
            pltpu.sync_copy(mag_hbm.at[i0 + k, pl.ds(base, _CH)], m_v[k])

        def ibody(j, icarry):
            sl = pl.ds(j * 16, 16)
            idx = idx_v[sl]
            for k in range(nimg):
                plsc.addupdate_scatter(h_v[k], [idx], m_v[k][sl])
            return icarry

        lax.fori_loop(0, _CH // 16, ibody, 0)
        return carry

    lax.fori_loop(0, _NCHUNK, cbody, 0)
    for k in range(nimg):
        pltpu.sync_copy(h_v[k], out_hbm.at[i0 + k])


def _sc_hist(mag_flat, rmap):
    nb = mag_flat.shape[0]
    nimg = nb // _NW
    mesh = plsc.VectorSubcoreMesh(
        core_axis_name="c", subcore_axis_name="s",
        num_cores=_NC, num_subcores=_NS)
    kern = functools.partial(
        pl.kernel,
        out_type=jax.ShapeDtypeStruct((nb, NBINS), jnp.float32),
        mesh=mesh,
        scratch_types=[pltpu.VMEM((_CH,), jnp.int32)]
        + [pltpu.VMEM((_CH,), jnp.float32) for _ in range(nimg)]
        + [pltpu.VMEM((NBINS,), jnp.float32) for _ in range(nimg)],
        compiler_params=pltpu.CompilerParams(needs_layout_passes=False),
    )(functools.partial(_sc_hist_body, nimg))
    return kern(mag_flat, rmap)


def _head1_body(sums_ref, invc_ref, m1_ref, b1_ref, out_ref):
    prof = sums_ref[:, :MAXR] * invc_ref[...]
    lg = jnp.log1p(prof)
    mn = jnp.min(lg, axis=1, keepdims=True)
    mx = jnp.max(lg, axis=1, keepdims=True)
    rng = mx - mn
    xn = jnp.where(rng > 0, (lg - mn) / rng, jnp.zeros_like(lg))
    h1 = jnp.dot(xn, m1_ref[...], preferred_element_type=jnp.float32)
    out_ref[...] = jnp.maximum(h1 + b1_ref[...], 0.0)


def _head1(sums, invc, m1, b1row):
    return pl.pallas_call(
        _head1_body,
        out_shape=jax.ShapeDtypeStruct((B, 16 * MAXR), jnp.float32),
    )(sums, invc, m1, b1row)


def _head2_body(xp_ref, w0_ref, w1_ref, w2_ref, b2_ref, mp_ref, ml_ref,
                out_ref):
    n = B * 128
    a = xp_ref[0:n]
    bm = xp_ref[1:n + 1]
    cm = xp_ref[2:n + 2]
    pprev = jnp.maximum(a[:, :16], a[:, 16:]) * mp_ref[...]
    pcent = jnp.maximum(bm[:, :16], bm[:, 16:])
    pnext = jnp.maximum(cm[:, :16], cm[:, 16:]) * ml_ref[...]
    h2 = (jnp.dot(pprev, w0_ref[...], preferred_element_type=jnp.float32)
          + jnp.dot(pcent, w1_ref[...], preferred_element_type=jnp.float32)
          + jnp.dot(pnext, w2_ref[...], preferred_element_type=jnp.float32))
    out_ref[...] = jnp.maximum(h2 + b2_ref[...], 0.0)


def _head2(xpad, w0, w1, w2, b2row, mp, ml):
    return pl.pallas_call(
        _head2_body,
        out_shape=jax.ShapeDtypeStruct((B * 128, 32), jnp.float32),
    )(xpad, w0, w1, w2, b2row, mp, ml)


def _head3_body(xp_ref, w0_ref, w1_ref, w2_ref, b3_ref, mp_ref, ml_ref,
                s_ref, wl_ref, bl_ref, out_ref):
    n = B * 64
    a = xp_ref[0:n]
    bm = xp_ref[1:n + 1]
    cm = xp_ref[2:n + 2]
    pprev = jnp.maximum(a[:, :32], a[:, 32:]) * mp_ref[...]
    pcent = jnp.maximum(bm[:, :32], bm[:, 32:])
    pnext = jnp.maximum(cm[:, :32], cm[:, 32:]) * ml_ref[...]
    h3 = (jnp.dot(pprev, w0_ref[...], preferred_element_type=jnp.float32)
          + jnp.dot(pcent, w1_ref[...], preferred_element_type=jnp.float32)
          + jnp.dot(pnext, w2_ref[...], preferred_element_type=jnp.float32))
    h3 = jnp.maximum(h3 + b3_ref[...], 0.0)
    proj = jnp.dot(h3, wl_ref[...], preferred_element_type=jnp.float32)
    out_ref[...] = (jnp.dot(s_ref[...], proj,
                            preferred_element_type=jnp.float32)
                    + bl_ref[...])


def _head3(xpad, w0, w1, w2, b3row, mp, ml, s, wlT, bl):
    return pl.pallas_call(
        _head3_body,
        out_shape=jax.ShapeDtypeStruct((B, 1), jnp.float32),
    )(xpad, w0, w1, w2, b3row, mp, ml, s, wlT, bl)


def kernel(x, W1, b1, W2, b2, W3, b3, Wl, bl):
    rmap = jnp.asarray(_RMAPH_NP)
    invc = jnp.asarray(_INVC_NP)

    mag = _fft_mag(x, jnp.asarray(_FRH_NP), jnp.asarray(_FIH_NP),
                   jnp.asarray(_F2R_NP), jnp.asarray(_F2I_NP),
                   jnp.asarray(_TWR_NP), jnp.asarray(_TWI_NP),
                   jnp.asarray(_WCOL_NP))
    sums = _sc_hist(mag.reshape(B, NPIXH), rmap)

    m1 = (jnp.zeros((MAXR, 16 * MAXR), jnp.float32)
          .at[jnp.asarray(_M1_ROWS), jnp.asarray(_M1_COLS)]
          .set(W1.reshape(-1)[jnp.asarray(_M1_WIDX)]))
    b1row = jnp.repeat(b1, MAXR).reshape(1, 16 * MAXR)

    h1 = _head1(sums, invc, m1, b1row)

    xpre = h1.reshape(B, 16, 128, 2).transpose(0, 2, 3, 1).reshape(B * 128, 32)
    xpad = jnp.pad(xpre, ((1, 1), (0, 0)))

    w2k = [W2[:, :, k].T for k in range(3)]
    b2row = b2.reshape(1, 32)
    h2 = _head2(xpad, w2k[0], w2k[1], w2k[2], b2row,
                jnp.asarray(_MP2_NP), jnp.asarray(_ML2_NP))

    xpre3 = h2.reshape(B, 64, 2, 32).reshape(B * 64, 64)
    xpad3 = jnp.pad(xpre3, ((1, 1), (0, 0)))

    w3k = [W3[:, :, k].T for k in range(3)]
    b3row = b3.reshape(1, 64)
    out = _head3(xpad3, w3k[0], w3k[1], w3k[2], b3row,
                 jnp.asarray(_MP3_NP), jnp.asarray(_ML3_NP),
                 jnp.asarray(_S_NP), Wl.T, bl.reshape(1, 1))
    return out


# speedup vs baseline: 1.8054x; 1.8054x over previous
import functools

import numpy as np
import jax
import jax.numpy as jnp
from jax import lax
from jax.experimental import pallas as pl
from jax.experimental.pallas import tpu as pltpu
from jax.experimental.pallas import tpu_sc as plsc

H = W = 512
B = 64
NPIX = H * W
MAXR = 256
NBINS = 512
NCOLH = 256
NPIXH = H * NCOLH


def _bitrev(n_bits):
    n = 1 << n_bits
    p = np.zeros(n, np.int64)
    for i in range(n):
        b = 0
        for k in range(n_bits):
            b |= ((i >> k) & 1) << (n_bits - 1 - k)
        p[i] = b
    return p


_PERM_NP = _bitrev(8)


def _dft_mats():
    j = np.arange(H, dtype=np.int64)
    jk = np.outer(j, j) % H
    ang = (2.0 * np.pi / H) * jk.astype(np.float64)
    fr = (np.cos(ang) / H).astype(np.float32)
    fi = (-np.sin(ang) / H).astype(np.float32)
    frh = np.ascontiguousarray(fr[:, :NCOLH][:, _PERM_NP])
    fih = np.ascontiguousarray(fi[:, :NCOLH][:, _PERM_NP])
    a = np.arange(256, dtype=np.int64)
    as2 = np.outer(a, a) % 256
    ang2 = (2.0 * np.pi / 256.0) * as2.astype(np.float64)
    f2r = (np.cos(ang2) / H).astype(np.float32)
    f2i = (-np.sin(ang2) / H).astype(np.float32)
    u = np.arange(256, dtype=np.float64)[:, None]
    wr = np.broadcast_to(np.cos(2.0 * np.pi * u / 512.0), (256, 256))
    wi = np.broadcast_to(-np.sin(2.0 * np.pi * u / 512.0), (256, 256))
    return (frh, fih, f2r, f2i,
            np.ascontiguousarray(wr).astype(np.float32),
            np.ascontiguousarray(wi).astype(np.float32))


_FRH_NP, _FIH_NP, _F2R_NP, _F2I_NP, _TWR_NP, _TWI_NP = _dft_mats()

_WCOL_NP = np.full((1, NCOLH), 2.0, np.float32)
_WCOL_NP[0, _PERM_NP == 0] = 1.0


def _radius_map():
    u = np.arange(H)
    d = ((u + H // 2) % H) - H // 2
    dy = d[:, None]
    dx = d[None, :]
    r = np.sqrt(dy * dy + dx * dx).astype(np.int64)
    return r.astype(np.int32)


_RMAP2D_NP = _radius_map()
_COUNTS_NP = np.bincount(
    _RMAP2D_NP.reshape(-1), minlength=NBINS).astype(np.float32)
_ROWORD_NP = np.concatenate([np.arange(0, H, 2), np.arange(1, H, 2)])
_RMAPH_NP = np.ascontiguousarray(
    _RMAP2D_NP[_ROWORD_NP][:, :NCOLH][:, _PERM_NP]).reshape(-1)
_INVC_NP = np.zeros((1, MAXR), np.float32)
_INVC_NP[0, :] = 1.0 / _COUNTS_NP[:MAXR]

_M1_ROWS, _M1_COLS, _M1_WIDX = [], [], []
for _o in range(16):
    for _t in range(MAXR):
        for _k in range(3):
            _s = _t + _k - 1
            if 0 <= _s < MAXR:
                _M1_ROWS.append(_s)
                _M1_COLS.append(_o * MAXR + _t)
                _M1_WIDX.append(_o * 3 + _k)
_M1_ROWS = np.asarray(_M1_ROWS, np.int32)
_M1_COLS = np.asarray(_M1_COLS, np.int32)
_M1_WIDX = np.asarray(_M1_WIDX, np.int32)


def _edge_masks(rows, period):
    t = np.arange(rows) % period
    mp = (t != 0).astype(np.float32).reshape(rows, 1)
    ml = (t != period - 1).astype(np.float32).reshape(rows, 1)
    return mp, ml


_MP2_NP, _ML2_NP = _edge_masks(B * 128, 128)
_MP3_NP, _ML3_NP = _edge_masks(B * 64, 64)

_S_NP = np.zeros((B, B * 64), np.float32)
for _b in range(B):
    _S_NP[_b, _b * 64:(_b + 1) * 64] = 1.0 / 64.0


def _fft_mag_body(x_ref, frh_ref, fih_ref, f2r_ref, f2i_ref, twr_ref,
                  twi_ref, w_ref, out_ref):
    r = x_ref[0, 0]
    g = x_ref[0, 1]
    b = x_ref[0, 2]
    gray = 0.2989 * r + 0.587 * g + 0.114 * b
    dot = functools.partial(jnp.dot, preferred_element_type=jnp.float32)
    zr = dot(gray, frh_ref[...])
    zi = dot(gray, fih_ref[...])
    ar = zr[0:256] + zr[256:512]
    ai = zi[0:256] + zi[256:512]
    dr = zr[0:256] - zr[256:512]
    di = zi[0:256] - zi[256:512]
    twr = twr_ref[...]
    twi = twi_ref[...]
    br = twr * dr - twi * di
    bi = twr * di + twi * dr
    f2r = f2r_ref[...]
    f2i = f2i_ref[...]
    yer = dot(f2r, ar) - dot(f2i, ai)
    yei = dot(f2r, ai) + dot(f2i, ar)
    yor = dot(f2r, br) - dot(f2i, bi)
    yoi = dot(f2r, bi) + dot(f2i, br)
    w = w_ref[...]
    out_ref[0, 0:256] = jnp.sqrt(yer * yer + yei * yei) * w
    out_ref[0, 256:512] = jnp.sqrt(yor * yor + yoi * yoi) * w


def _fft_mag(x, frh, fih, f2r, f2i, twr, twi, wcol):
    nb = x.shape[0]
    return pl.pallas_call(
        _fft_mag_body,
        grid=(nb,),
        in_specs=[
            pl.BlockSpec((1, 3, H, W), lambda i: (i, 0, 0, 0)),
            pl.BlockSpec((H, NCOLH), lambda i: (0, 0)),
            pl.BlockSpec((H, NCOLH), lambda i: (0, 0)),
            pl.BlockSpec((256, 256), lambda i: (0, 0)),
            pl.BlockSpec((256, 256), lambda i: (0, 0)),
            pl.BlockSpec((256, 256), lambda i: (0, 0)),
            pl.BlockSpec((256, 256), lambda i: (0, 0)),
            pl.BlockSpec((1, NCOLH), lambda i: (0, 0)),
        ],
        out_specs=pl.BlockSpec((1, H, NCOLH), lambda i: (i, 0, 0)),
        out_shape=jax.ShapeDtypeStruct((nb, H, NCOLH), jnp.float32),
        compiler_params=pltpu.CompilerParams(
            dimension_semantics=("arbitrary",)),
    )(x, frh, fih, f2r, f2i, twr, twi, wcol)


_NC, _NS = 2, 16
_NW = _NC * _NS
_CH = 16384
_NCHUNK = NPIXH // _CH
_IMGS_PER_W = B // _NW


def _sc_hist_body(nimg, mag_hbm, rmap_hbm, out_hbm, idx_v, *vbufs):
    wid = lax.axis_index("s") * _NC + lax.axis_index("c")
    i0 = wid * nimg
    m_v = vbufs[:nimg]
    h_v = vbufs[nimg:]

    zero = jnp.zeros((16,), jnp.float32)

    def zbody(j, carry):
        for k in range(nimg):
            h_v[k][pl.ds(j * 16, 16)] = zero
        return carry

    lax.fori_loop(0, NBINS // 16, zbody, 0)

    def cbody(c, carry):
        base = c * _CH
        pltpu.sync_copy(rmap_hbm.at[pl.ds(base, _CH)], idx_v)
        for k in range(nimg):
            pltpu.sync_copy(mag_hbm.at[i0 + k, pl.ds(base, _CH)], m_v[k])

        def ibody(j, icarry):
            sl = pl.ds(j * 16, 16)
            idx = idx_v[sl]
            for k in range(nimg):
                plsc.addupdate_scatter(h_v[k], [idx], m_v[k][sl])
            return icarry

        lax.fori_loop(0, _CH // 16, ibody, 0)
        return carry

    lax.fori_loop(0, _NCHUNK, cbody, 0)
    for k in range(nimg):
        pltpu.sync_copy(h_v[k], out_hbm.at[i0 + k])


def _sc_hist(mag_flat, rmap):
    nb = mag_flat.shape[0]
    nimg = nb // _NW
    mesh = plsc.VectorSubcoreMesh(
        core_axis_name="c", subcore_axis_name="s",
        num_cores=_NC, num_subcores=_NS)
    kern = functools.partial(
        pl.kernel,
        out_type=jax.ShapeDtypeStruct((nb, NBINS), jnp.float32),
        mesh=mesh,
        scratch_types=[pltpu.VMEM((_CH,), jnp.int32)]
        + [pltpu.VMEM((_CH,), jnp.float32) for _ in range(nimg)]
        + [pltpu.VMEM((NBINS,), jnp.float32) for _ in range(nimg)],
        compiler_params=pltpu.CompilerParams(needs_layout_passes=False),
    )(functools.partial(_sc_hist_body, nimg))
    return kern(mag_flat, rmap)


def _head1_body(sums_ref, invc_ref, m1_ref, b1_ref, out_ref):
    prof = sums_ref[:, :MAXR] * invc_ref[...]
    lg = jnp.log1p(prof)
    mn = jnp.min(lg, axis=1, keepdims=True)
    mx = jnp.max(lg, axis=1, keepdims=True)
    rng = mx - mn
    xn = jnp.where(rng > 0, (lg - mn) / rng, jnp.zeros_like(lg))
    h1 = jnp.dot(xn, m1_ref[...], preferred_element_type=jnp.float32)
    out_ref[...] = jnp.maximum(h1 + b1_ref[...], 0.0)


def _head1(sums, invc, m1, b1row):
    return pl.pallas_call(
        _head1_body,
        out_shape=jax.ShapeDtypeStruct((B, 16 * MAXR), jnp.float32),
    )(sums, invc, m1, b1row)


def _head2_body(xp_ref, w0_ref, w1_ref, w2_ref, b2_ref, mp_ref, ml_ref,
                out_ref):
    n = B * 128
    a = xp_ref[0:n]
    bm = xp_ref[1:n + 1]
    cm = xp_ref[2:n + 2]
    pprev = jnp.maximum(a[:, :16], a[:, 16:]) * mp_ref[...]
    pcent = jnp.maximum(bm[:, :16], bm[:, 16:])
    pnext = jnp.maximum(cm[:, :16], cm[:, 16:]) * ml_ref[...]
    h2 = (jnp.dot(pprev, w0_ref[...], preferred_element_type=jnp.float32)
          + jnp.dot(pcent, w1_ref[...], preferred_element_type=jnp.float32)
          + jnp.dot(pnext, w2_ref[...], preferred_element_type=jnp.float32))
    out_ref[...] = jnp.maximum(h2 + b2_ref[...], 0.0)


def _head2(xpad, w0, w1, w2, b2row, mp, ml):
    return pl.pallas_call(
        _head2_body,
        out_shape=jax.ShapeDtypeStruct((B * 128, 32), jnp.float32),
    )(xpad, w0, w1, w2, b2row, mp, ml)


def _head3_body(xp_ref, w0_ref, w1_ref, w2_ref, b3_ref, mp_ref, ml_ref,
                s_ref, wl_ref, bl_ref, out_ref):
    n = B * 64
    a = xp_ref[0:n]
    bm = xp_ref[1:n + 1]
    cm = xp_ref[2:n + 2]
    pprev = jnp.maximum(a[:, :32], a[:, 32:]) * mp_ref[...]
    pcent = jnp.maximum(bm[:, :32], bm[:, 32:])
    pnext = jnp.maximum(cm[:, :32], cm[:, 32:]) * ml_ref[...]
    h3 = (jnp.dot(pprev, w0_ref[...], preferred_element_type=jnp.float32)
          + jnp.dot(pcent, w1_ref[...], preferred_element_type=jnp.float32)
          + jnp.dot(pnext, w2_ref[...], preferred_element_type=jnp.float32))
    h3 = jnp.maximum(h3 + b3_ref[...], 0.0)
    proj = jnp.dot(h3, wl_ref[...], preferred_element_type=jnp.float32)
    out_ref[...] = (jnp.dot(s_ref[...], proj,
                            preferred_element_type=jnp.float32)
                    + bl_ref[...])


def _head3(xpad, w0, w1, w2, b3row, mp, ml, s, wlT, bl):
    return pl.pallas_call(
        _head3_body,
        out_shape=jax.ShapeDtypeStruct((B, 1), jnp.float32),
    )(xpad, w0, w1, w2, b3row, mp, ml, s, wlT, bl)


def kernel(x, W1, b1, W2, b2, W3, b3, Wl, bl):
    rmap = jnp.asarray(_RMAPH_NP)
    invc = jnp.asarray(_INVC_NP)

    mag = _fft_mag(x, jnp.asarray(_FRH_NP), jnp.asarray(_FIH_NP),
                   jnp.asarray(_F2R_NP), jnp.asarray(_F2I_NP),
                   jnp.asarray(_TWR_NP), jnp.asarray(_TWI_NP),
                   jnp.asarray(_WCOL_NP))
    sums = _sc_hist(mag.reshape(B, NPIXH), rmap)

    m1 = (jnp.zeros((MAXR, 16 * MAXR), jnp.float32)
          .at[jnp.asarray(_M1_ROWS), jnp.asarray(_M1_COLS)]
          .set(W1.reshape(-1)[jnp.asarray(_M1_WIDX)]))
    b1row = jnp.repeat(b1, MAXR).reshape(1, 16 * MAXR)

    h1 = _head1(sums, invc, m1, b1row)

    xpre = h1.reshape(B, 16, 128, 2).transpose(0, 2, 3, 1).reshape(B * 128, 32)
    xpad = jnp.pad(xpre, ((1, 1), (0, 0)))

    w2k = [W2[:, :, k].T for k in range(3)]
    b2row = b2.reshape(1, 32)
    h2 = _head2(xpad, w2k[0], w2k[1], w2k[2], b2row,
                jnp.asarray(_MP2_NP), jnp.asarray(_ML2_NP))

    xpre3 = h2.reshape(B, 64, 2, 32).reshape(B * 64, 64)
    xpad3 = jnp.pad(xpre3, ((1, 1), (0, 0)))

    w3k = [W3[:, :, k].T for k in range(3)]
    b3row = b3.reshape(1, 64)
    out = _head3(xpad3, w3k[0], w3k[1], w3k[2], b3row,
                 jnp.asarray(_MP3_NP), jnp.asarray(_ML3_NP),
                 jnp.asarray(_S_NP), Wl.T, bl.reshape(1, 1))
    return out


# speedup vs baseline: 2.0302x; 1.1245x over previous
import functools

import numpy as np
import jax
import jax.numpy as jnp
from jax import lax
from jax.experimental import pallas as pl
from jax.experimental.pallas import tpu as pltpu
from jax.experimental.pallas import tpu_sc as plsc

H = W = 512
B = 64
NPIX = H * W
MAXR = 256
NBINS = 512
NCOLH = 256
NROWF = 256
NPIXF = NROWF * NCOLH


def _bitrev(n_bits):
    n = 1 << n_bits
    p = np.zeros(n, np.int64)
    for i in range(n):
        b = 0
        for k in range(n_bits):
            b |= ((i >> k) & 1) << (n_bits - 1 - k)
        p[i] = b
    return p


_PERM_NP = _bitrev(8)


def _dft_mats():
    j = np.arange(H, dtype=np.int64)
    jk = np.outer(j, j) % H
    ang = (2.0 * np.pi / H) * jk.astype(np.float64)
    fr = (np.cos(ang) / H).astype(np.float32)
    fi = (-np.sin(ang) / H).astype(np.float32)
    frh = np.ascontiguousarray(fr[:, :NCOLH][:, _PERM_NP])
    fih = np.ascontiguousarray(fi[:, :NCOLH][:, _PERM_NP])
    a = np.arange(256, dtype=np.int64)
    as2 = np.outer(a, a) % 256
    ang2 = (2.0 * np.pi / 256.0) * as2.astype(np.float64)
    f2r = (np.cos(ang2) / H).astype(np.float32)
    f2i = (-np.sin(ang2) / H).astype(np.float32)
    u = np.arange(256, dtype=np.float64)[:, None]
    wr = np.broadcast_to(np.cos(2.0 * np.pi * u / 512.0), (256, 256))
    wi = np.broadcast_to(-np.sin(2.0 * np.pi * u / 512.0), (256, 256))
    return (frh, fih, f2r, f2i,
            np.ascontiguousarray(wr).astype(np.float32),
            np.ascontiguousarray(wi).astype(np.float32))


_FRH_NP, _FIH_NP, _F2R_NP, _F2I_NP, _TWR_NP, _TWI_NP = _dft_mats()

_WCOL_NP = np.full((1, NCOLH), 2.0, np.float32)
_WCOL_NP[0, _PERM_NP == 0] = 1.0


def _radius_map():
    u = np.arange(H)
    d = ((u + H // 2) % H) - H // 2
    dy = d[:, None]
    dx = d[None, :]
    r = np.sqrt(dy * dy + dx * dx).astype(np.int64)
    return r.astype(np.int32)


_RMAP2D_NP = _radius_map()
_COUNTS_NP = np.bincount(
    _RMAP2D_NP.reshape(-1), minlength=NBINS).astype(np.float32)
_DYF_NP = np.concatenate([2 * np.arange(128), 2 * np.arange(128) + 1])
_DVF_NP = _PERM_NP
_RMAPH_NP = np.ascontiguousarray(
    np.sqrt(_DYF_NP[:, None] ** 2 + _DVF_NP[None, :] ** 2)
    .astype(np.int64).astype(np.int32)).reshape(-1)

_AE_NP = np.zeros((128, 256), np.float32)
_AO_NP = np.zeros((128, 256), np.float32)
for _k in range(128):
    _AE_NP[_k, _k] = 1.0
    if _k > 0:
        _AE_NP[_k, 256 - _k] += 1.0
    _AO_NP[_k, _k] = 1.0
    _AO_NP[_k, 255 - _k] += 1.0
_INVC_NP = np.zeros((1, MAXR), np.float32)
_INVC_NP[0, :] = 1.0 / _COUNTS_NP[:MAXR]

_M1_ROWS, _M1_COLS, _M1_WIDX = [], [], []
for _o in range(16):
    for _t in range(MAXR):
        for _k in range(3):
            _s = _t + _k - 1
            if 0 <= _s < MAXR:
                _M1_ROWS.append(_s)
                _M1_COLS.append(_o * MAXR + _t)
                _M1_WIDX.append(_o * 3 + _k)
_M1_ROWS = np.asarray(_M1_ROWS, np.int32)
_M1_COLS = np.asarray(_M1_COLS, np.int32)
_M1_WIDX = np.asarray(_M1_WIDX, np.int32)


def _edge_masks(rows, period):
    t = np.arange(rows) % period
    mp = (t != 0).astype(np.float32).reshape(rows, 1)
    ml = (t != period - 1).astype(np.float32).reshape(rows, 1)
    return mp, ml


_MP2_NP, _ML2_NP = _edge_masks(B * 128, 128)
_MP3_NP, _ML3_NP = _edge_masks(B * 64, 64)

_S_NP = np.zeros((B, B * 64), np.float32)
for _b in range(B):
    _S_NP[_b, _b * 64:(_b + 1) * 64] = 1.0 / 64.0


def _fft_mag_body(x_ref, frh_ref, fih_ref, f2r_ref, f2i_ref, twr_ref,
                  twi_ref, w_ref, ae_ref, ao_ref, out_ref):
    r = x_ref[0, 0]
    g = x_ref[0, 1]
    b = x_ref[0, 2]
    gray = 0.2989 * r + 0.587 * g + 0.114 * b
    dot = functools.partial(jnp.dot, preferred_element_type=jnp.float32)
    zr = dot(gray, frh_ref[...])
    zi = dot(gray, fih_ref[...])
    ar = zr[0:256] + zr[256:512]
    ai = zi[0:256] + zi[256:512]
    dr = zr[0:256] - zr[256:512]
    di = zi[0:256] - zi[256:512]
    twr = twr_ref[...]
    twi = twi_ref[...]
    br = twr * dr - twi * di
    bi = twr * di + twi * dr
    f2r = f2r_ref[...]
    f2i = f2i_ref[...]
    yer = dot(f2r, ar) - dot(f2i, ai)
    yei = dot(f2r, ai) + dot(f2i, ar)
    yor = dot(f2r, br) - dot(f2i, bi)
    yoi = dot(f2r, bi) + dot(f2i, br)
    w = w_ref[...]
    ye_mag = jnp.sqrt(yer * yer + yei * yei)
    yo_mag = jnp.sqrt(yor * yor + yoi * yoi)
    out_ref[0, 0:128] = dot(ae_ref[...], ye_mag) * w
    out_ref[0, 128:256] = dot(ao_ref[...], yo_mag) * w


def _fft_mag(x, frh, fih, f2r, f2i, twr, twi, wcol, ae, ao):
    nb = x.shape[0]
    return pl.pallas_call(
        _fft_mag_body,
        grid=(nb,),
        in_specs=[
            pl.BlockSpec((1, 3, H, W), lambda i: (i, 0, 0, 0)),
            pl.BlockSpec((H, NCOLH), lambda i: (0, 0)),
            pl.BlockSpec((H, NCOLH), lambda i: (0, 0)),
            pl.BlockSpec((256, 256), lambda i: (0, 0)),
            pl.BlockSpec((256, 256), lambda i: (0, 0)),
            pl.BlockSpec((256, 256), lambda i: (0, 0)),
            pl.BlockSpec((256, 256), lambda i: (0, 0)),
            pl.BlockSpec((1, NCOLH), lambda i: (0, 0)),
            pl.BlockSpec((128, 256), lambda i: (0, 0)),
            pl.BlockSpec((128, 256), lambda i: (0, 0)),
        ],
        out_specs=pl.BlockSpec((1, NROWF, NCOLH), lambda i: (i, 0, 0)),
        out_shape=jax.ShapeDtypeStruct((nb, NROWF, NCOLH), jnp.float32),
        compiler_params=pltpu.CompilerParams(
            dimension_semantics=("arbitrary",)),
    )(x, frh, fih, f2r, f2i, twr, twi, wcol, ae, ao)


_NC, _NS = 2, 16
_NW = _NC * _NS
_CH = 16384
_NCHUNK = NPIXF // _CH
_IMGS_PER_W = B // _NW


def _sc_hist_body(nimg, mag_hbm, rmap_hbm, out_hbm, idx_v, *vbufs):
    wid = lax.axis_index("s") * _NC + lax.axis_index("c")
    i0 = wid * nimg
    m_v = vbufs[:nimg]
    h_v = vbufs[nimg:]

    zero = jnp.zeros((16,), jnp.float32)

    def zbody(j, carry):
        for k in range(nimg):
            h_v[k][pl.ds(j * 16, 16)] = zero
        return carry

    lax.fori_loop(0, NBINS // 16, zbody, 0)

    def cbody(c, carry):
        base = c * _CH
        pltpu.sync_copy(rmap_hbm.at[pl.ds(base, _CH)], idx_v)
        for k in range(nimg):
            pltpu.sync_copy(mag_hbm.at[i0 + k, pl.ds(base, _CH)], m_v[k])

        def ibody(j, icarry):
            sl = pl.ds(j * 16, 16)
            idx = idx_v[sl]
            for k in range(nimg):
                plsc.addupdate_scatter(h_v[k], [idx], m_v[k][sl])
            return icarry

        lax.fori_loop(0, _CH // 16, ibody, 0)
        return carry

    lax.fori_loop(0, _NCHUNK, cbody, 0)
    for k in range(nimg):
        pltpu.sync_copy(h_v[k], out_hbm.at[i0 + k])


def _sc_hist(mag_flat, rmap):
    nb = mag_flat.shape[0]
    nimg = nb // _NW
    mesh = plsc.VectorSubcoreMesh(
        core_axis_name="c", subcore_axis_name="s",
        num_cores=_NC, num_subcores=_NS)
    kern = functools.partial(
        pl.kernel,
        out_type=jax.ShapeDtypeStruct((nb, NBINS), jnp.float32),
        mesh=mesh,
        scratch_types=[pltpu.VMEM((_CH,), jnp.int32)]
        + [pltpu.VMEM((_CH,), jnp.float32) for _ in range(nimg)]
        + [pltpu.VMEM((NBINS,), jnp.float32) for _ in range(nimg)],
        compiler_params=pltpu.CompilerParams(needs_layout_passes=False),
    )(functools.partial(_sc_hist_body, nimg))
    return kern(mag_flat, rmap)


def _head1_body(sums_ref, invc_ref, m1_ref, b1_ref, out_ref):
    prof = sums_ref[:, :MAXR] * invc_ref[...]
    lg = jnp.log1p(prof)
    mn = jnp.min(lg, axis=1, keepdims=True)
    mx = jnp.max(lg, axis=1, keepdims=True)
    rng = mx - mn
    xn = jnp.where(rng > 0, (lg - mn) / rng, jnp.zeros_like(lg))
    h1 = jnp.dot(xn, m1_ref[...], preferred_element_type=jnp.float32)
    out_ref[...] = jnp.maximum(h1 + b1_ref[...], 0.0)


def _head1(sums, invc, m1, b1row):
    return pl.pallas_call(
        _head1_body,
        out_shape=jax.ShapeDtypeStruct((B, 16 * MAXR), jnp.float32),
    )(sums, invc, m1, b1row)


def _head2_body(xp_ref, w0_ref, w1_ref, w2_ref, b2_ref, mp_ref, ml_ref,
                out_ref):
    n = B * 128
    a = xp_ref[0:n]
    bm = xp_ref[1:n + 1]
    cm = xp_ref[2:n + 2]
    pprev = jnp.maximum(a[:, :16], a[:, 16:]) * mp_ref[...]
    pcent = jnp.maximum(bm[:, :16], bm[:, 16:])
    pnext = jnp.maximum(cm[:, :16], cm[:, 16:]) * ml_ref[...]
    h2 = (jnp.dot(pprev, w0_ref[...], preferred_element_type=jnp.float32)
          + jnp.dot(pcent, w1_ref[...], preferred_element_type=jnp.float32)
          + jnp.dot(pnext, w2_ref[...], preferred_element_type=jnp.float32))
    out_ref[...] = jnp.maximum(h2 + b2_ref[...], 0.0)


def _head2(xpad, w0, w1, w2, b2row, mp, ml):
    return pl.pallas_call(
        _head2_body,
        out_shape=jax.ShapeDtypeStruct((B * 128, 32), jnp.float32),
    )(xpad, w0, w1, w2, b2row, mp, ml)


def _head3_body(xp_ref, w0_ref, w1_ref, w2_ref, b3_ref, mp_ref, ml_ref,
                s_ref, wl_ref, bl_ref, out_ref):
    n = B * 64
    a = xp_ref[0:n]
    bm = xp_ref[1:n + 1]
    cm = xp_ref[2:n + 2]
    pprev = jnp.maximum(a[:, :32], a[:, 32:]) * mp_ref[...]
    pcent = jnp.maximum(bm[:, :32], bm[:, 32:])
    pnext = jnp.maximum(cm[:, :32], cm[:, 32:]) * ml_ref[...]
    h3 = (jnp.dot(pprev, w0_ref[...], preferred_element_type=jnp.float32)
          + jnp.dot(pcent, w1_ref[...], preferred_element_type=jnp.float32)
          + jnp.dot(pnext, w2_ref[...], preferred_element_type=jnp.float32))
    h3 = jnp.maximum(h3 + b3_ref[...], 0.0)
    proj = jnp.dot(h3, wl_ref[...], preferred_element_type=jnp.float32)
    out_ref[...] = (jnp.dot(s_ref[...], proj,
                            preferred_element_type=jnp.float32)
                    + bl_ref[...])


def _head3(xpad, w0, w1, w2, b3row, mp, ml, s, wlT, bl):
    return pl.pallas_call(
        _head3_body,
        out_shape=jax.ShapeDtypeStruct((B, 1), jnp.float32),
    )(xpad, w0, w1, w2, b3row, mp, ml, s, wlT, bl)


def kernel(x, W1, b1, W2, b2, W3, b3, Wl, bl):
    rmap = jnp.asarray(_RMAPH_NP)
    invc = jnp.asarray(_INVC_NP)

    mag = _fft_mag(x, jnp.asarray(_FRH_NP), jnp.asarray(_FIH_NP),
                   jnp.asarray(_F2R_NP), jnp.asarray(_F2I_NP),
                   jnp.asarray(_TWR_NP), jnp.asarray(_TWI_NP),
                   jnp.asarray(_WCOL_NP), jnp.asarray(_AE_NP),
                   jnp.asarray(_AO_NP))
    sums = _sc_hist(mag.reshape(B, NPIXF), rmap)

    m1 = (jnp.zeros((MAXR, 16 * MAXR), jnp.float32)
          .at[jnp.asarray(_M1_ROWS), jnp.asarray(_M1_COLS)]
          .set(W1.reshape(-1)[jnp.asarray(_M1_WIDX)]))
    b1row = jnp.repeat(b1, MAXR).reshape(1, 16 * MAXR)

    h1 = _head1(sums, invc, m1, b1row)

    xpre = h1.reshape(B, 16, 128, 2).transpose(0, 2, 3, 1).reshape(B * 128, 32)
    xpad = jnp.pad(xpre, ((1, 1), (0, 0)))

    w2k = [W2[:, :, k].T for k in range(3)]
    b2row = b2.reshape(1, 32)
    h2 = _head2(xpad, w2k[0], w2k[1], w2k[2], b2row,
                jnp.asarray(_MP2_NP), jnp.asarray(_ML2_NP))

    xpre3 = h2.reshape(B, 64, 2, 32).reshape(B * 64, 64)
    xpad3 = jnp.pad(xpre3, ((1, 1), (0, 0)))

    w3k = [W3[:, :, k].T for k in range(3)]
    b3row = b3.reshape(1, 64)
    out = _head3(xpad3, w3k[0], w3k[1], w3k[2], b3row,
                 jnp.asarray(_MP3_NP), jnp.asarray(_ML3_NP),
                 jnp.asarray(_S_NP), Wl.T, bl.reshape(1, 1))
    return out


# speedup vs baseline: 2.1416x; 1.0549x over previous
import functools

import numpy as np
import jax
import jax.numpy as jnp
from jax import lax
from jax.experimental import pallas as pl
from jax.experimental.pallas import tpu as pltpu
from jax.experimental.pallas import tpu_sc as plsc

H = W = 512
B = 64
NPIX = H * W
MAXR = 256
NBINS = 512
NCOLH = 256
NROWF = 256
NPIXF = NROWF * NCOLH


def _bitrev(n_bits):
    n = 1 << n_bits
    p = np.zeros(n, np.int64)
    for i in range(n):
        b = 0
        for k in range(n_bits):
            b |= ((i >> k) & 1) << (n_bits - 1 - k)
        p[i] = b
    return p


_PERM_NP = _bitrev(8)


def _dft_mats():
    j = np.arange(H, dtype=np.int64)
    jk = np.outer(j, j) % H
    ang = (2.0 * np.pi / H) * jk.astype(np.float64)
    fr = (np.cos(ang) / H).astype(np.float32)
    fi = (-np.sin(ang) / H).astype(np.float32)
    frh = np.ascontiguousarray(fr[:, :NCOLH][:, _PERM_NP])
    fih = np.ascontiguousarray(fi[:, :NCOLH][:, _PERM_NP])
    a = np.arange(256, dtype=np.int64)
    as2 = np.outer(a, a) % 256
    ang2 = (2.0 * np.pi / 256.0) * as2.astype(np.float64)
    f2r = (np.cos(ang2) / H).astype(np.float32)
    f2i = (-np.sin(ang2) / H).astype(np.float32)
    u = np.arange(256, dtype=np.float64)[:, None]
    wr = np.broadcast_to(np.cos(2.0 * np.pi * u / 512.0), (256, 256))
    wi = np.broadcast_to(-np.sin(2.0 * np.pi * u / 512.0), (256, 256))
    return (frh, fih, f2r, f2i,
            np.ascontiguousarray(wr).astype(np.float32),
            np.ascontiguousarray(wi).astype(np.float32))


_FRH_NP, _FIH_NP, _F2R_NP, _F2I_NP, _TWR_NP, _TWI_NP = _dft_mats()

_WCOL_NP = np.full((1, NCOLH), 2.0, np.float32)
_WCOL_NP[0, _PERM_NP == 0] = 1.0


def _radius_map():
    u = np.arange(H)
    d = ((u + H // 2) % H) - H // 2
    dy = d[:, None]
    dx = d[None, :]
    r = np.sqrt(dy * dy + dx * dx).astype(np.int64)
    return r.astype(np.int32)


_RMAP2D_NP = _radius_map()
_COUNTS_NP = np.bincount(
    _RMAP2D_NP.reshape(-1), minlength=NBINS).astype(np.float32)
_DYF_NP = np.concatenate([2 * np.arange(128), 2 * np.arange(128) + 1])
_DVF_NP = _PERM_NP
_RMAPH_NP = np.ascontiguousarray(
    np.sqrt(_DYF_NP[:, None] ** 2 + _DVF_NP[None, :] ** 2)
    .astype(np.int64).astype(np.int32)).reshape(-1)

_AE_NP = np.zeros((128, 256), np.float32)
_AO_NP = np.zeros((128, 256), np.float32)
for _k in range(128):
    _AE_NP[_k, _k] = 1.0
    if _k > 0:
        _AE_NP[_k, 256 - _k] += 1.0
    _AO_NP[_k, _k] = 1.0
    _AO_NP[_k, 255 - _k] += 1.0
_INVC_NP = np.zeros((1, MAXR), np.float32)
_INVC_NP[0, :] = 1.0 / _COUNTS_NP[:MAXR]

_M1_ROWS, _M1_COLS, _M1_WIDX = [], [], []
for _o in range(16):
    for _t in range(MAXR):
        for _k in range(3):
            _s = _t + _k - 1
            if 0 <= _s < MAXR:
                _M1_ROWS.append(_s)
                _M1_COLS.append(_o * MAXR + _t)
                _M1_WIDX.append(_o * 3 + _k)
_M1_ROWS = np.asarray(_M1_ROWS, np.int32)
_M1_COLS = np.asarray(_M1_COLS, np.int32)
_M1_WIDX = np.asarray(_M1_WIDX, np.int32)


def _edge_masks(rows, period):
    t = np.arange(rows) % period
    mp = (t != 0).astype(np.float32).reshape(rows, 1)
    ml = (t != period - 1).astype(np.float32).reshape(rows, 1)
    return mp, ml


_MP2_NP, _ML2_NP = _edge_masks(B * 128, 128)
_MP3_NP, _ML3_NP = _edge_masks(B * 64, 64)

_S_NP = np.zeros((B, B * 64), np.float32)
for _b in range(B):
    _S_NP[_b, _b * 64:(_b + 1) * 64] = 1.0 / 64.0


def _fft_mag_body(x_ref, frh_ref, fih_ref, f2r_ref, f2i_ref, twr_ref,
                  twi_ref, w_ref, ae_ref, ao_ref, out_ref):
    r = x_ref[0, 0]
    g = x_ref[0, 1]
    b = x_ref[0, 2]
    gray = 0.2989 * r + 0.587 * g + 0.114 * b
    dot = functools.partial(jnp.dot, preferred_element_type=jnp.float32)
    zr = dot(gray, frh_ref[...])
    zi = dot(gray, fih_ref[...])
    ar = zr[0:256] + zr[256:512]
    ai = zi[0:256] + zi[256:512]
    dr = zr[0:256] - zr[256:512]
    di = zi[0:256] - zi[256:512]
    twr = twr_ref[...]
    twi = twi_ref[...]
    br = twr * dr - twi * di
    bi = twr * di + twi * dr
    f2r = f2r_ref[...]
    f2i = f2i_ref[...]
    yer = dot(f2r, ar) - dot(f2i, ai)
    yei = dot(f2r, ai) + dot(f2i, ar)
    yor = dot(f2r, br) - dot(f2i, bi)
    yoi = dot(f2r, bi) + dot(f2i, br)
    w = w_ref[...]
    ye_mag = jnp.sqrt(yer * yer + yei * yei)
    yo_mag = jnp.sqrt(yor * yor + yoi * yoi)
    out_ref[0, 0:128] = dot(ae_ref[...], ye_mag) * w
    out_ref[0, 128:256] = dot(ao_ref[...], yo_mag) * w


def _fft_mag(x, frh, fih, f2r, f2i, twr, twi, wcol, ae, ao):
    nb = x.shape[0]
    return pl.pallas_call(
        _fft_mag_body,
        grid=(nb,),
        in_specs=[
            pl.BlockSpec((1, 3, H, W), lambda i: (i, 0, 0, 0)),
            pl.BlockSpec((H, NCOLH), lambda i: (0, 0)),
            pl.BlockSpec((H, NCOLH), lambda i: (0, 0)),
            pl.BlockSpec((256, 256), lambda i: (0, 0)),
            pl.BlockSpec((256, 256), lambda i: (0, 0)),
            pl.BlockSpec((256, 256), lambda i: (0, 0)),
            pl.BlockSpec((256, 256), lambda i: (0, 0)),
            pl.BlockSpec((1, NCOLH), lambda i: (0, 0)),
            pl.BlockSpec((128, 256), lambda i: (0, 0)),
            pl.BlockSpec((128, 256), lambda i: (0, 0)),
        ],
        out_specs=pl.BlockSpec((1, NROWF, NCOLH), lambda i: (i, 0, 0)),
        out_shape=jax.ShapeDtypeStruct((nb, NROWF, NCOLH), jnp.float32),
        compiler_params=pltpu.CompilerParams(
            dimension_semantics=("arbitrary",)),
    )(x, frh, fih, f2r, f2i, twr, twi, wcol, ae, ao)


_NC, _NS = 2, 16
_NW = _NC * _NS
_RCH = 64
_NCHUNK = NROWF // _RCH
_IMGS_PER_W = B // _NW


def _sc_hist_body(nimg, mag_hbm, rmap_hbm, out_hbm, idx_v, *vbufs):
    wid = lax.axis_index("s") * _NC + lax.axis_index("c")
    i0 = wid * nimg
    m_v = vbufs[:nimg]
    h_v = vbufs[nimg:]

    zero = jnp.zeros((16,), jnp.float32)

    def zbody(j, carry):
        for k in range(nimg):
            h_v[k][pl.ds(j * 16, 16)] = zero
        return carry

    lax.fori_loop(0, NBINS // 16, zbody, 0)

    def cbody(c, carry):
        r0 = c * _RCH
        pltpu.sync_copy(rmap_hbm.at[pl.ds(r0, _RCH), :], idx_v)
        for k in range(nimg):
            pltpu.sync_copy(mag_hbm.at[i0 + k, pl.ds(r0, _RCH), :], m_v[k])

        def ibody(r, icarry):
            for t in range(NCOLH // 16):
                sl = pl.ds(t * 16, 16)
                idx = idx_v[r, sl]
                for k in range(nimg):
                    plsc.addupdate_scatter(h_v[k], [idx], m_v[k][r, sl])
            return icarry

        lax.fori_loop(0, _RCH, ibody, 0)
        return carry

    lax.fori_loop(0, _NCHUNK, cbody, 0)
    for k in range(nimg):
        pltpu.sync_copy(h_v[k], out_hbm.at[i0 + k])


def _sc_hist(mag, rmap):
    nb = mag.shape[0]
    nimg = nb // _NW
    mesh = plsc.VectorSubcoreMesh(
        core_axis_name="c", subcore_axis_name="s",
        num_cores=_NC, num_subcores=_NS)
    kern = functools.partial(
        pl.kernel,
        out_type=jax.ShapeDtypeStruct((nb, NBINS), jnp.float32),
        mesh=mesh,
        scratch_types=[pltpu.VMEM((_RCH, NCOLH), jnp.int32)]
        + [pltpu.VMEM((_RCH, NCOLH), jnp.float32) for _ in range(nimg)]
        + [pltpu.VMEM((NBINS,), jnp.float32) for _ in range(nimg)],
        compiler_params=pltpu.CompilerParams(needs_layout_passes=False),
    )(functools.partial(_sc_hist_body, nimg))
    return kern(mag, rmap)


def _head1_body(sums_ref, invc_ref, m1_ref, b1_ref, out_ref):
    prof = sums_ref[:, :MAXR] * invc_ref[...]
    lg = jnp.log1p(prof)
    mn = jnp.min(lg, axis=1, keepdims=True)
    mx = jnp.max(lg, axis=1, keepdims=True)
    rng = mx - mn
    xn = jnp.where(rng > 0, (lg - mn) / rng, jnp.zeros_like(lg))
    h1 = jnp.dot(xn, m1_ref[...], preferred_element_type=jnp.float32)
    out_ref[...] = jnp.maximum(h1 + b1_ref[...], 0.0)


def _head1(sums, invc, m1, b1row):
    return pl.pallas_call(
        _head1_body,
        out_shape=jax.ShapeDtypeStruct((B, 16 * MAXR), jnp.float32),
    )(sums, invc, m1, b1row)


def _head2_body(xp_ref, w0_ref, w1_ref, w2_ref, b2_ref, mp_ref, ml_ref,
                out_ref):
    n = B * 128
    a = xp_ref[0:n]
    bm = xp_ref[1:n + 1]
    cm = xp_ref[2:n + 2]
    pprev = jnp.maximum(a[:, :16], a[:, 16:]) * mp_ref[...]
    pcent = jnp.maximum(bm[:, :16], bm[:, 16:])
    pnext = jnp.maximum(cm[:, :16], cm[:, 16:]) * ml_ref[...]
    h2 = (jnp.dot(pprev, w0_ref[...], preferred_element_type=jnp.float32)
          + jnp.dot(pcent, w1_ref[...], preferred_element_type=jnp.float32)
          + jnp.dot(pnext, w2_ref[...], preferred_element_type=jnp.float32))
    out_ref[...] = jnp.maximum(h2 + b2_ref[...], 0.0)


def _head2(xpad, w0, w1, w2, b2row, mp, ml):
    return pl.pallas_call(
        _head2_body,
        out_shape=jax.ShapeDtypeStruct((B * 128, 32), jnp.float32),
    )(xpad, w0, w1, w2, b2row, mp, ml)


def _head3_body(xp_ref, w0_ref, w1_ref, w2_ref, b3_ref, mp_ref, ml_ref,
                s_ref, wl_ref, bl_ref, out_ref):
    n = B * 64
    a = xp_ref[0:n]
    bm = xp_ref[1:n + 1]
    cm = xp_ref[2:n + 2]
    pprev = jnp.maximum(a[:, :32], a[:, 32:]) * mp_ref[...]
    pcent = jnp.maximum(bm[:, :32], bm[:, 32:])
    pnext = jnp.maximum(cm[:, :32], cm[:, 32:]) * ml_ref[...]
    h3 = (jnp.dot(pprev, w0_ref[...], preferred_element_type=jnp.float32)
          + jnp.dot(pcent, w1_ref[...], preferred_element_type=jnp.float32)
          + jnp.dot(pnext, w2_ref[...], preferred_element_type=jnp.float32))
    h3 = jnp.maximum(h3 + b3_ref[...], 0.0)
    proj = jnp.dot(h3, wl_ref[...], preferred_element_type=jnp.float32)
    out_ref[...] = (jnp.dot(s_ref[...], proj,
                            preferred_element_type=jnp.float32)
                    + bl_ref[...])


def _head3(xpad, w0, w1, w2, b3row, mp, ml, s, wlT, bl):
    return pl.pallas_call(
        _head3_body,
        out_shape=jax.ShapeDtypeStruct((B, 1), jnp.float32),
    )(xpad, w0, w1, w2, b3row, mp, ml, s, wlT, bl)


def kernel(x, W1, b1, W2, b2, W3, b3, Wl, bl):
    rmap = jnp.asarray(_RMAPH_NP.reshape(NROWF, NCOLH))
    invc = jnp.asarray(_INVC_NP)

    mag = _fft_mag(x, jnp.asarray(_FRH_NP), jnp.asarray(_FIH_NP),
                   jnp.asarray(_F2R_NP), jnp.asarray(_F2I_NP),
                   jnp.asarray(_TWR_NP), jnp.asarray(_TWI_NP),
                   jnp.asarray(_WCOL_NP), jnp.asarray(_AE_NP),
                   jnp.asarray(_AO_NP))
    sums = _sc_hist(mag, rmap)

    m1 = (jnp.zeros((MAXR, 16 * MAXR), jnp.float32)
          .at[jnp.asarray(_M1_ROWS), jnp.asarray(_M1_COLS)]
          .set(W1.reshape(-1)[jnp.asarray(_M1_WIDX)]))
    b1row = jnp.repeat(b1, MAXR).reshape(1, 16 * MAXR)

    h1 = _head1(sums, invc, m1, b1row)

    xpre = h1.reshape(B, 16, 128, 2).transpose(0, 2, 3, 1).reshape(B * 128, 32)
    xpad = jnp.pad(xpre, ((1, 1), (0, 0)))

    w2k = [W2[:, :, k].T for k in range(3)]
    b2row = b2.reshape(1, 32)
    h2 = _head2(xpad, w2k[0], w2k[1], w2k[2], b2row,
                jnp.asarray(_MP2_NP), jnp.asarray(_ML2_NP))

    xpre3 = h2.reshape(B, 64, 2, 32).reshape(B * 64, 64)
    xpad3 = jnp.pad(xpre3, ((1, 1), (0, 0)))

    w3k = [W3[:, :, k].T for k in range(3)]
    b3row = b3.reshape(1, 64)
    out = _head3(xpad3, w3k[0], w3k[1], w3k[2], b3row,
                 jnp.asarray(_MP3_NP), jnp.asarray(_ML3_NP),
                 jnp.asarray(_S_NP), Wl.T, bl.reshape(1, 1))
    return out
